# baseline (speedup 1.0000x reference)
import functools

import jax
import jax.numpy as jnp
from jax import lax
from jax.experimental import pallas as pl
from jax.experimental.pallas import tpu as pltpu
from jax.experimental.pallas import tpu_sc as plsc

_NW = 32
_K = 8


def kernel(x, pos_emb):
    nb, ns, nd = x.shape
    rows_pw = ns // _NW
    mesh = plsc.VectorSubcoreMesh(core_axis_name="c", subcore_axis_name="s")

    @functools.partial(
        pl.kernel,
        out_type=jax.ShapeDtypeStruct((nb, ns, nd), jnp.float32),
        mesh=mesh,
        scratch_types=[
            pltpu.VMEM((_K, nd), jnp.float32),
            pltpu.VMEM((_K, nd), jnp.float32),
        ],
    )
    def sc_add(x_hbm, pos_hbm, out_hbm, pos_v, x_v):
        wid = lax.axis_index("s") * 2 + lax.axis_index("c")
        s0 = wid * rows_pw

        def chunk_body(c, carry):
            row = s0 + c * _K
            pltpu.sync_copy(pos_hbm.at[pl.ds(row, _K)], pos_v)

            def batch_body(b, inner):
                pltpu.sync_copy(x_hbm.at[b, pl.ds(row, _K)], x_v)
                for r in range(_K):
                    for j in range(nd // 16):
                        sl = pl.ds(j * 16, 16)
                        plsc.addupdate(x_v.at[r, sl], pos_v[r, sl])
                pltpu.sync_copy(x_v, out_hbm.at[b, pl.ds(row, _K)])
                return inner

            return lax.fori_loop(0, nb, batch_body, carry)

        lax.fori_loop(0, rows_pw // _K, chunk_body, 0)

    return sc_add(x, pos_emb)


# speedup vs baseline: 1.5198x; 1.5198x over previous
import functools

import jax
import jax.numpy as jnp
from jax import lax
from jax.experimental import pallas as pl
from jax.experimental.pallas import tpu as pltpu
from jax.experimental.pallas import tpu_sc as plsc

_NW = 32
_K = 8


def kernel(x, pos_emb):
    nb, ns, nd = x.shape
    rows_pw = ns // _NW
    nsteps = (rows_pw // _K) * nb
    mesh = plsc.VectorSubcoreMesh(core_axis_name="c", subcore_axis_name="s")

    @functools.partial(
        pl.kernel,
        out_type=jax.ShapeDtypeStruct((nb, ns, nd), jnp.float32),
        mesh=mesh,
        scratch_types=[
            pltpu.VMEM((_K, nd), jnp.float32),
            pltpu.VMEM((2, _K, nd), jnp.float32),
            pltpu.SemaphoreType.DMA((2,)),
            pltpu.SemaphoreType.DMA((2,)),
        ],
    )
    def sc_add(x_hbm, pos_hbm, out_hbm, pos_v, xb_v, in_sem, out_sem):
        wid = lax.axis_index("s") * 2 + lax.axis_index("c")
        s0 = wid * rows_pw

        def in_copy(step, p):
            c = step // nb
            b = step % nb
            row = s0 + c * _K
            return pltpu.make_async_copy(
                x_hbm.at[b, pl.ds(row, _K)], xb_v.at[p], in_sem.at[p])

        def out_copy(step, p):
            c = step // nb
            b = step % nb
            row = s0 + c * _K
            return pltpu.make_async_copy(
                xb_v.at[p], out_hbm.at[b, pl.ds(row, _K)], out_sem.at[p])

        in_copy(0, 0).start()

        def step_body(s, carry):
            p = s % 2
            q = (s + 1) % 2
            b = s % nb

            in_copy(s, p).wait()

            @pl.when(b == 0)
            def _():
                row = s0 + (s // nb) * _K
                pltpu.sync_copy(pos_hbm.at[pl.ds(row, _K)], pos_v)

            @pl.when(s + 1 < nsteps)
            def _():
                @pl.when(s >= 1)
                def _():
                    out_copy(s - 1, q).wait()
                in_copy(s + 1, q).start()

            for r in range(_K):
                for j in range(nd // 16):
                    sl = pl.ds(j * 16, 16)
                    plsc.addupdate(xb_v.at[p, r, sl], pos_v[r, sl])

            out_copy(s, p).start()
            return carry

        lax.fori_loop(0, nsteps, step_body, 0)
        out_copy(nsteps - 2, nsteps % 2).wait()
        out_copy(nsteps - 1, (nsteps - 1) % 2).wait()

    return sc_add(x, pos_emb)


# speedup vs baseline: 1.8309x; 1.2047x over previous
import functools

import jax
import jax.numpy as jnp
from jax import lax
from jax.experimental import pallas as pl
from jax.experimental.pallas import tpu as pltpu
from jax.experimental.pallas import tpu_sc as plsc

_NW = 32
_K = 16


def kernel(x, pos_emb):
    nb, ns, nd = x.shape
    rows_pw = ns // _NW
    nsteps = (rows_pw // _K) * nb
    mesh = plsc.VectorSubcoreMesh(core_axis_name="c", subcore_axis_name="s")

    @functools.partial(
        pl.kernel,
        out_type=jax.ShapeDtypeStruct((nb, ns, nd), jnp.float32),
        mesh=mesh,
        scratch_types=[
            pltpu.VMEM((_K, nd), jnp.float32),
            pltpu.VMEM((2, _K, nd), jnp.float32),
            pltpu.SemaphoreType.DMA((2,)),
            pltpu.SemaphoreType.DMA((2,)),
        ],
    )
    def sc_add(x_hbm, pos_hbm, out_hbm, pos_v, xb_v, in_sem, out_sem):
        wid = lax.axis_index("s") * 2 + lax.axis_index("c")
        s0 = wid * rows_pw

        def in_copy(step, p):
            c = step // nb
            b = step % nb
            row = s0 + c * _K
            return pltpu.make_async_copy(
                x_hbm.at[b, pl.ds(row, _K)], xb_v.at[p], in_sem.at[p])

        def out_copy(step, p):
            c = step // nb
            b = step % nb
            row = s0 + c * _K
            return pltpu.make_async_copy(
                xb_v.at[p], out_hbm.at[b, pl.ds(row, _K)], out_sem.at[p])

        in_copy(0, 0).start()

        def step_body(s, carry):
            p = s % 2
            q = (s + 1) % 2
            b = s % nb

            in_copy(s, p).wait()

            @pl.when(b == 0)
            def _():
                row = s0 + (s // nb) * _K
                pltpu.sync_copy(pos_hbm.at[pl.ds(row, _K)], pos_v)

            @pl.when(s + 1 < nsteps)
            def _():
                @pl.when(s >= 1)
                def _():
                    out_copy(s - 1, q).wait()
                in_copy(s + 1, q).start()

            def row_body(r, rcarry):
                for j in range(nd // 16):
                    sl = pl.ds(j * 16, 16)
                    plsc.addupdate(xb_v.at[p, r, sl], pos_v[r, sl])
                return rcarry

            lax.fori_loop(0, _K, row_body, 0)

            out_copy(s, p).start()
            return carry

        lax.fori_loop(0, nsteps, step_body, 0)
        out_copy(nsteps - 2, nsteps % 2).wait()
        out_copy(nsteps - 1, (nsteps - 1) % 2).wait()

    return sc_add(x, pos_emb)


# speedup vs baseline: 5.7150x; 3.1214x over previous
import jax
import jax.numpy as jnp
from jax.experimental import pallas as pl
from jax.experimental.pallas import tpu as pltpu

_BS = 1024


def _add_body(x_ref, pos_ref, out_ref):
    out_ref[0] = x_ref[0] + pos_ref[...]


def kernel(x, pos_emb):
    batch, seq, d = x.shape
    grid = (seq // _BS, batch)
    return pl.pallas_call(
        _add_body,
        grid=grid,
        in_specs=[
            pl.BlockSpec((1, _BS, d), lambda i, j: (j, i, 0)),
            pl.BlockSpec((_BS, d), lambda i, j: (i, 0)),
        ],
        out_specs=pl.BlockSpec((1, _BS, d), lambda i, j: (j, i, 0)),
        out_shape=jax.ShapeDtypeStruct(x.shape, x.dtype),
        compiler_params=pltpu.CompilerParams(
            vmem_limit_bytes=120 * 1024 * 1024,
        ),
    )(x, pos_emb)


# speedup vs baseline: 5.7168x; 1.0003x over previous
import jax
import jax.numpy as jnp
from jax.experimental import pallas as pl

_BS = 1024


def _add_body(x_ref, pos_ref, out_ref):
    out_ref[0] = x_ref[0] + pos_ref[...]


def kernel(x, pos_emb):
    batch, seq, d = x.shape
    grid = (seq // _BS, batch)
    return pl.pallas_call(
        _add_body,
        grid=grid,
        in_specs=[
            pl.BlockSpec((1, _BS, d), lambda i, j: (j, i, 0)),
            pl.BlockSpec((_BS, d), lambda i, j: (i, 0)),
        ],
        out_specs=pl.BlockSpec((1, _BS, d), lambda i, j: (j, i, 0)),
        out_shape=jax.ShapeDtypeStruct(x.shape, x.dtype),
    )(x, pos_emb)
